# baseline (speedup 1.0000x reference)
import functools
import numpy as np
import jax
import jax.numpy as jnp
from jax import lax
from jax.experimental import pallas as pl
from jax.experimental.pallas import tpu as pltpu, tpu_sc as plsc


def _build_idx_list():
    num_candidates = 16
    indices = [0, 1, 2, 3, 4, 5, 6, 7, 8]
    base_idx = 9
    for i in range(num_candidates - 1):
        indices += [6, 7, base_idx + i]
    indices += [0, 3, 6, 1, 4, 7, 2, 5, 8]
    for i in range(num_candidates - 1):
        indices += [2, 5, base_idx + i]
    return indices


_IDX = np.array(_build_idx_list(), dtype=np.int32)

_B, _N, _S, _D = 8, 24, 512, 128
_CPS = 8
_ROWW = (_S // _CPS) * _D
_NROWS = _B * 108 * _CPS
_NW = 32
_RPW = _NROWS // _NW
_K = _CPS


def _src_rows():
    b = np.arange(_B, dtype=np.int32)
    c = np.arange(_CPS, dtype=np.int32)
    src = (b[:, None, None] * _N + _IDX[None, :, None]) * _CPS + c[None, None, :]
    return src.reshape(_NW, _RPW // _K, _K)


_SRC = _src_rows()


def kernel(x):
    x_flat = x.reshape(_B * _N * _CPS, _ROWW)
    src = jnp.asarray(_SRC)
    mesh = plsc.VectorSubcoreMesh(core_axis_name="c", subcore_axis_name="s")

    @functools.partial(
        pl.kernel,
        mesh=mesh,
        out_type=jax.ShapeDtypeStruct((_NROWS, _ROWW), jnp.float32),
        scratch_types=[
            pltpu.VMEM((_RPW // _K, _K), jnp.int32),
            pltpu.VMEM((_K, _ROWW), jnp.float32),
            pltpu.SemaphoreType.DMA,
        ],
    )
    def k(x_hbm, src_hbm, out_hbm, idx_v, buf_v, sem):
        wid = lax.axis_index("s") * 2 + lax.axis_index("c")
        pltpu.sync_copy(src_hbm.at[wid], idx_v)
        base = wid * _RPW

        def body(g, carry):
            pltpu.async_copy(x_hbm.at[idx_v.at[g]], buf_v, sem).wait()
            pltpu.sync_copy(buf_v, out_hbm.at[pl.ds(base + g * _K, _K)])
            return carry

        lax.fori_loop(0, _RPW // _K, body, 0)

    out = k(x_flat, src)
    return out.reshape(_B, 36, 3, _S, _D)
